# Initial kernel scaffold; baseline (speedup 1.0000x reference)
import jax
import jax.numpy as jnp
from jax.experimental import pallas as pl


def kernel(edges, node_features, edge_features, node_enc, edge_enc, edge_proc, node_proc, dec):
    raise NotImplementedError("write your pallas kernel here")


# speedup vs baseline: 5.6482x; 5.6482x over previous
import jax
import jax.numpy as jnp
from jax import lax
from jax.experimental import pallas as pl
from jax.experimental.pallas import tpu as pltpu
from jax.experimental.pallas import tpu_sc as plsc

_N = 10000
_E = 320000
_DF = 128
_DE = 16
_H = 128

_NC = 2
_NS = 16
_NW = _NC * _NS
_EPW = _E // _NW
_CH = 80
_NCHUNK = _EPW // _CH
_NPAD = 10240
_RPN = _NPAD // _NS

_BE = 3200

_PREC = lax.Precision.HIGHEST
_f32 = jnp.float32


def _tc_node_pre(nf, w1n, b1n, w2n, b2n, wb, wc, w2e, wa, b2e, b1p):
    def body(nf_r, w1n_r, b1n_r, w2n_r, b2n_r, wb_r, wc_r, w2e_r, wa_r,
             b2e_r, b1p_r, nfo_o, p0_o, p1_o, wfe_o, bfe_o):
        h = jnp.maximum(jnp.dot(nf_r[...], w1n_r[...], precision=_PREC) + b1n_r[...], 0.0)
        nfo = jnp.dot(h, w2n_r[...], precision=_PREC) + b2n_r[...]
        nfo_o[...] = nfo
        p0_o[...] = jnp.dot(nfo, wb_r[...], precision=_PREC)
        p1_o[...] = jnp.dot(nfo, wc_r[...], precision=_PREC)
        wfe_o[...] = jnp.dot(w2e_r[...], wa_r[...], precision=_PREC)
        bfe_o[...] = jnp.dot(b2e_r[...], wa_r[...], precision=_PREC) + b1p_r[...]

    return pl.pallas_call(
        body,
        out_shape=[
            jax.ShapeDtypeStruct((_N, _H), _f32),
            jax.ShapeDtypeStruct((_N, _H), _f32),
            jax.ShapeDtypeStruct((_N, _H), _f32),
            jax.ShapeDtypeStruct((_H, _H), _f32),
            jax.ShapeDtypeStruct((1, _H), _f32),
        ],
    )(nf, w1n, b1n, w2n, b2n, wb, wc, w2e, wa, b2e, b1p)


def _sc_gather(p0, p1, i0, i1):
    mesh = plsc.VectorSubcoreMesh(core_axis_name="c", subcore_axis_name="s")

    def gather_k(p0_hbm, p1_hbm, i0_hbm, i1_hbm, g0_hbm, g1_hbm,
                 i0_v, i1_v, r0_v, r1_v, s0, s1):
        wid = lax.axis_index("s") * _NC + lax.axis_index("c")
        base = wid * _EPW

        def chunk(k, carry):
            off = base + k * _CH
            pltpu.sync_copy(i0_hbm.at[pl.ds(off, _CH)], i0_v)
            pltpu.sync_copy(i1_hbm.at[pl.ds(off, _CH)], i1_v)
            c0 = pltpu.async_copy(p0_hbm.at[i0_v], r0_v, s0)
            c1 = pltpu.async_copy(p1_hbm.at[i1_v], r1_v, s1)
            c0.wait()
            c1.wait()
            pltpu.sync_copy(r0_v, g0_hbm.at[pl.ds(off, _CH)])
            pltpu.sync_copy(r1_v, g1_hbm.at[pl.ds(off, _CH)])
            return carry

        lax.fori_loop(0, _NCHUNK, chunk, 0)

    call = pl.kernel(
        gather_k,
        mesh=mesh,
        out_type=[
            jax.ShapeDtypeStruct((_E, _H), _f32),
            jax.ShapeDtypeStruct((_E, _H), _f32),
        ],
        scratch_types=[
            pltpu.VMEM((_CH,), jnp.int32),
            pltpu.VMEM((_CH,), jnp.int32),
            pltpu.VMEM((_CH, _H), _f32),
            pltpu.VMEM((_CH, _H), _f32),
            pltpu.SemaphoreType.DMA,
            pltpu.SemaphoreType.DMA,
        ],
    )
    return call(p0, p1, i0, i1)


def _tc_edge(ef, g0, g1, w1e, b1e, wfe, bfe, w2p, b2p):
    nblk = _E // _BE

    def body(ef_r, g0_r, g1_r, w1e_r, b1e_r, wfe_r, bfe_r, w2p_r, b2p_r, m_o):
        h = jnp.maximum(jnp.dot(ef_r[...], w1e_r[...], precision=_PREC) + b1e_r[...], 0.0)
        z = jnp.maximum(
            jnp.dot(h, wfe_r[...], precision=_PREC) + bfe_r[...] + g0_r[...] + g1_r[...],
            0.0)
        m_o[...] = jnp.dot(z, w2p_r[...], precision=_PREC) + b2p_r[...]

    def full(shape):
        return pl.BlockSpec(shape, lambda i: (0, 0))

    return pl.pallas_call(
        body,
        grid=(nblk,),
        in_specs=[
            pl.BlockSpec((_BE, _DE), lambda i: (i, 0)),
            pl.BlockSpec((_BE, _H), lambda i: (i, 0)),
            pl.BlockSpec((_BE, _H), lambda i: (i, 0)),
            full((_DE, _H)),
            full((1, _H)),
            full((_H, _H)),
            full((1, _H)),
            full((_H, _H)),
            full((1, _H)),
        ],
        out_specs=pl.BlockSpec((_BE, _H), lambda i: (i, 0)),
        out_shape=jax.ShapeDtypeStruct((_E, _H), _f32),
    )(ef, g0, g1, w1e, b1e, wfe, bfe, w2p, b2p)


def _sc_scatter(m, i0, zeros):
    mesh = plsc.VectorSubcoreMesh(core_axis_name="c", subcore_axis_name="s")

    def scatter_k(m_hbm, i0_hbm, z_hbm, out_hbm, i_v, r_v, acc_sh):
        c = lax.axis_index("c")
        s = lax.axis_index("s")
        pltpu.sync_copy(z_hbm, acc_sh.at[pl.ds(s * _RPN, _RPN)])
        plsc.subcore_barrier()

        base = (c * _NS + s) * _EPW

        def chunk(k, carry):
            off = base + k * _CH
            pltpu.sync_copy(m_hbm.at[pl.ds(off, _CH)], r_v)
            pltpu.sync_copy(i0_hbm.at[pl.ds(off, _CH)], i_v)
            pltpu.sync_copy(r_v, acc_sh.at[i_v], add=True)
            return carry

        lax.fori_loop(0, _NCHUNK, chunk, 0)
        plsc.subcore_barrier()
        pltpu.sync_copy(acc_sh.at[pl.ds(s * _RPN, _RPN)],
                        out_hbm.at[pl.ds(c * _NPAD + s * _RPN, _RPN)])

    call = pl.kernel(
        scatter_k,
        mesh=mesh,
        out_type=jax.ShapeDtypeStruct((_NC * _NPAD, _H), _f32),
        scratch_types=[
            pltpu.VMEM((_CH,), jnp.int32),
            pltpu.VMEM((_CH, _H), _f32),
            pltpu.VMEM_SHARED((_NPAD, _H), _f32),
        ],
    )
    return call(m, i0, zeros)


def _tc_node_post(nf, nfo, a0, a1, wqa, wqb, b1q, w2q, b2q, w1d, b1d, w2d, b2d):
    def body(nf_r, nfo_r, a0_r, a1_r, wqa_r, wqb_r, b1q_r, w2q_r, b2q_r,
             w1d_r, b1d_r, w2d_r, b2d_r, out_o):
        agg = a0_r[...] + a1_r[...]
        u = jnp.maximum(
            jnp.dot(nfo_r[...], wqa_r[...], precision=_PREC)
            + jnp.dot(agg, wqb_r[...], precision=_PREC) + b1q_r[...], 0.0)
        u = jnp.dot(u, w2q_r[...], precision=_PREC) + b2q_r[...]
        v = jnp.maximum(jnp.dot(u, w1d_r[...], precision=_PREC) + b1d_r[...], 0.0)
        v = jnp.dot(v, w2d_r[...], precision=_PREC) + b2d_r[...]
        out_o[...] = nf_r[...] + v

    return pl.pallas_call(
        body,
        out_shape=jax.ShapeDtypeStruct((_N, _DF), _f32),
    )(nf, nfo, a0, a1, wqa, wqb, b1q, w2q, b2q, w1d, b1d, w2d, b2d)


def kernel(edges, node_features, edge_features, node_enc, edge_enc, edge_proc,
           node_proc, dec):
    e0 = edges[0, 0].astype(jnp.int32)
    e1 = edges[0, 1].astype(jnp.int32)
    nf = node_features[0]
    ef = edge_features[0]
    w1n, b1n, w2n, b2n = node_enc
    w1e, b1e, w2e, b2e = edge_enc
    w1p, b1p, w2p, b2p = edge_proc
    w1q, b1q, w2q, b2q = node_proc
    w1d, b1d, w2d, b2d = dec

    wa, wb, wc = w1p[:_H], w1p[_H:2 * _H], w1p[2 * _H:]
    wqa, wqb = w1q[:_H], w1q[_H:]

    def r(b):
        return b.reshape(1, -1)

    nfo, p0, p1, wfe, bfe = _tc_node_pre(
        nf, w1n, r(b1n), w2n, r(b2n), wb, wc, w2e, wa, r(b2e), r(b1p))
    g0, g1 = _sc_gather(p0, p1, e0, e1)
    m = _tc_edge(ef, g0, g1, w1e, r(b1e), wfe, bfe, w2p, r(b2p))
    zeros = jnp.zeros((_RPN, _H), _f32)
    aggp = _sc_scatter(m, e0, zeros)
    out = _tc_node_post(
        nf, nfo, aggp[:_N], aggp[_NPAD:_NPAD + _N], wqa, wqb, r(b1q), w2q, r(b2q),
        w1d, r(b1d), w2d, r(b2d))
    return out[None]


# speedup vs baseline: 8.0497x; 1.4252x over previous
import jax
import jax.numpy as jnp
from jax import lax
from jax.experimental import pallas as pl
from jax.experimental.pallas import tpu as pltpu
from jax.experimental.pallas import tpu_sc as plsc

_N = 10000
_E = 320000
_DF = 128
_DE = 16
_H = 128

_NC = 2
_NS = 16
_NW = _NC * _NS
_EPW = _E // _NW
_CH = 80
_NCHUNK = _EPW // _CH
_NPAD = 10240
_RPN = _NPAD // _NS

_BE = 3200

_PREC = lax.Precision.HIGHEST
_EPREC = lax.Precision.DEFAULT
_f32 = jnp.float32


def _tc_node_pre(nf, w1n, b1n, w2n, b2n, wb, wc, w2e, wa, b2e, b1p):
    def body(nf_r, w1n_r, b1n_r, w2n_r, b2n_r, wb_r, wc_r, w2e_r, wa_r,
             b2e_r, b1p_r, nfo_o, p0_o, p1_o, wfe_o, bfe_o):
        h = jnp.maximum(jnp.dot(nf_r[...], w1n_r[...], precision=_PREC) + b1n_r[...], 0.0)
        nfo = jnp.dot(h, w2n_r[...], precision=_PREC) + b2n_r[...]
        nfo_o[...] = nfo
        p0_o[...] = jnp.dot(nfo, wb_r[...], precision=_PREC)
        p1_o[...] = jnp.dot(nfo, wc_r[...], precision=_PREC)
        wfe_o[...] = jnp.dot(w2e_r[...], wa_r[...], precision=_PREC)
        bfe_o[...] = jnp.dot(b2e_r[...], wa_r[...], precision=_PREC) + b1p_r[...]

    return pl.pallas_call(
        body,
        out_shape=[
            jax.ShapeDtypeStruct((_N, _H), _f32),
            jax.ShapeDtypeStruct((_N, _H), _f32),
            jax.ShapeDtypeStruct((_N, _H), _f32),
            jax.ShapeDtypeStruct((_H, _H), _f32),
            jax.ShapeDtypeStruct((1, _H), _f32),
        ],
    )(nf, w1n, b1n, w2n, b2n, wb, wc, w2e, wa, b2e, b1p)


def _sc_gather(p0, p1, i0, i1):
    mesh = plsc.VectorSubcoreMesh(core_axis_name="c", subcore_axis_name="s")

    def gather_k(p0_hbm, p1_hbm, i0_hbm, i1_hbm, g0_hbm, g1_hbm,
                 i0_v, i1_v, r0_v, r1_v, s0, s1):
        wid = lax.axis_index("s") * _NC + lax.axis_index("c")
        base = wid * _EPW

        def chunk(k, carry):
            off = base + k * _CH
            pltpu.sync_copy(i0_hbm.at[pl.ds(off, _CH)], i0_v)
            pltpu.sync_copy(i1_hbm.at[pl.ds(off, _CH)], i1_v)
            c0 = pltpu.async_copy(p0_hbm.at[i0_v], r0_v, s0)
            c1 = pltpu.async_copy(p1_hbm.at[i1_v], r1_v, s1)
            c0.wait()
            c1.wait()
            pltpu.sync_copy(r0_v, g0_hbm.at[pl.ds(off, _CH)])
            pltpu.sync_copy(r1_v, g1_hbm.at[pl.ds(off, _CH)])
            return carry

        lax.fori_loop(0, _NCHUNK, chunk, 0)

    call = pl.kernel(
        gather_k,
        mesh=mesh,
        out_type=[
            jax.ShapeDtypeStruct((_E, _H), _f32),
            jax.ShapeDtypeStruct((_E, _H), _f32),
        ],
        scratch_types=[
            pltpu.VMEM((_CH,), jnp.int32),
            pltpu.VMEM((_CH,), jnp.int32),
            pltpu.VMEM((_CH, _H), _f32),
            pltpu.VMEM((_CH, _H), _f32),
            pltpu.SemaphoreType.DMA,
            pltpu.SemaphoreType.DMA,
        ],
    )
    return call(p0, p1, i0, i1)


def _tc_edge(ef, g0, g1, w1e, b1e, wfe, bfe, w2p, b2p):
    nblk = _E // _BE

    def body(ef_r, g0_r, g1_r, w1e_r, b1e_r, wfe_r, bfe_r, w2p_r, b2p_r, m_o):
        h = jnp.maximum(jnp.dot(ef_r[...], w1e_r[...], precision=_EPREC) + b1e_r[...], 0.0)
        z = jnp.maximum(
            jnp.dot(h, wfe_r[...], precision=_EPREC) + bfe_r[...] + g0_r[...] + g1_r[...],
            0.0)
        m_o[...] = jnp.dot(z, w2p_r[...], precision=_EPREC) + b2p_r[...]

    def full(shape):
        return pl.BlockSpec(shape, lambda i: (0, 0))

    return pl.pallas_call(
        body,
        grid=(nblk,),
        in_specs=[
            pl.BlockSpec((_BE, _DE), lambda i: (i, 0)),
            pl.BlockSpec((_BE, _H), lambda i: (i, 0)),
            pl.BlockSpec((_BE, _H), lambda i: (i, 0)),
            full((_DE, _H)),
            full((1, _H)),
            full((_H, _H)),
            full((1, _H)),
            full((_H, _H)),
            full((1, _H)),
        ],
        out_specs=pl.BlockSpec((_BE, _H), lambda i: (i, 0)),
        out_shape=jax.ShapeDtypeStruct((_E, _H), _f32),
    )(ef, g0, g1, w1e, b1e, wfe, bfe, w2p, b2p)


def _sc_scatter(m, i0, zeros):
    mesh = plsc.VectorSubcoreMesh(core_axis_name="c", subcore_axis_name="s")

    def scatter_k(m_hbm, i0_hbm, z_hbm, out_hbm, i_v, r_v, acc_sh):
        c = lax.axis_index("c")
        s = lax.axis_index("s")
        pltpu.sync_copy(z_hbm, acc_sh.at[pl.ds(s * _RPN, _RPN)])
        plsc.subcore_barrier()

        base = (c * _NS + s) * _EPW

        def chunk(k, carry):
            off = base + k * _CH
            pltpu.sync_copy(m_hbm.at[pl.ds(off, _CH)], r_v)
            pltpu.sync_copy(i0_hbm.at[pl.ds(off, _CH)], i_v)
            pltpu.sync_copy(r_v, acc_sh.at[i_v], add=True)
            return carry

        lax.fori_loop(0, _NCHUNK, chunk, 0)
        plsc.subcore_barrier()
        pltpu.sync_copy(acc_sh.at[pl.ds(s * _RPN, _RPN)],
                        out_hbm.at[pl.ds(c * _NPAD + s * _RPN, _RPN)])

    call = pl.kernel(
        scatter_k,
        mesh=mesh,
        out_type=jax.ShapeDtypeStruct((_NC * _NPAD, _H), _f32),
        scratch_types=[
            pltpu.VMEM((_CH,), jnp.int32),
            pltpu.VMEM((_CH, _H), _f32),
            pltpu.VMEM_SHARED((_NPAD, _H), _f32),
        ],
    )
    return call(m, i0, zeros)


def _tc_node_post(nf, nfo, a0, a1, wqa, wqb, b1q, w2q, b2q, w1d, b1d, w2d, b2d):
    def body(nf_r, nfo_r, a0_r, a1_r, wqa_r, wqb_r, b1q_r, w2q_r, b2q_r,
             w1d_r, b1d_r, w2d_r, b2d_r, out_o):
        agg = a0_r[...] + a1_r[...]
        u = jnp.maximum(
            jnp.dot(nfo_r[...], wqa_r[...], precision=_PREC)
            + jnp.dot(agg, wqb_r[...], precision=_PREC) + b1q_r[...], 0.0)
        u = jnp.dot(u, w2q_r[...], precision=_PREC) + b2q_r[...]
        v = jnp.maximum(jnp.dot(u, w1d_r[...], precision=_PREC) + b1d_r[...], 0.0)
        v = jnp.dot(v, w2d_r[...], precision=_PREC) + b2d_r[...]
        out_o[...] = nf_r[...] + v

    return pl.pallas_call(
        body,
        out_shape=jax.ShapeDtypeStruct((_N, _DF), _f32),
    )(nf, nfo, a0, a1, wqa, wqb, b1q, w2q, b2q, w1d, b1d, w2d, b2d)


def kernel(edges, node_features, edge_features, node_enc, edge_enc, edge_proc,
           node_proc, dec):
    e0 = edges[0, 0].astype(jnp.int32)
    e1 = edges[0, 1].astype(jnp.int32)
    nf = node_features[0]
    ef = edge_features[0]
    w1n, b1n, w2n, b2n = node_enc
    w1e, b1e, w2e, b2e = edge_enc
    w1p, b1p, w2p, b2p = edge_proc
    w1q, b1q, w2q, b2q = node_proc
    w1d, b1d, w2d, b2d = dec

    wa, wb, wc = w1p[:_H], w1p[_H:2 * _H], w1p[2 * _H:]
    wqa, wqb = w1q[:_H], w1q[_H:]

    def r(b):
        return b.reshape(1, -1)

    nfo, p0, p1, wfe, bfe = _tc_node_pre(
        nf, w1n, r(b1n), w2n, r(b2n), wb, wc, w2e, wa, r(b2e), r(b1p))
    g0, g1 = _sc_gather(p0, p1, e0, e1)
    m = _tc_edge(ef, g0, g1, w1e, r(b1e), wfe, bfe, w2p, r(b2p))
    zeros = jnp.zeros((_RPN, _H), _f32)
    aggp = _sc_scatter(m, e0, zeros)
    out = _tc_node_post(
        nf, nfo, aggp[:_N], aggp[_NPAD:_NPAD + _N], wqa, wqb, r(b1q), w2q, r(b2q),
        w1d, r(b1d), w2d, r(b2d))
    return out[None]


# speedup vs baseline: 12.3615x; 1.5357x over previous
import jax
import jax.numpy as jnp
from jax import lax
from jax.experimental import pallas as pl
from jax.experimental.pallas import tpu as pltpu
from jax.experimental.pallas import tpu_sc as plsc

_N = 10000
_E = 320000
_DF = 128
_DE = 16
_H = 128

_NC = 2
_NS = 16
_NW = _NC * _NS
_EPW = _E // _NW
_CH = 80
_NCHUNK = _EPW // _CH
_NPAD = 10240
_RPN = _NPAD // _NS

_BE = 3200

_PREC = lax.Precision.HIGHEST
_EPREC = lax.Precision.DEFAULT
_f32 = jnp.float32


def _tc_node_pre(nf, w1n, b1n, w2n, b2n, wb, wc, w2e, wa, b2e, b1p):
    def body(nf_r, w1n_r, b1n_r, w2n_r, b2n_r, wb_r, wc_r, w2e_r, wa_r,
             b2e_r, b1p_r, nfo_o, p0_o, p1_o, wfe_o, bfe_o):
        h = jnp.maximum(jnp.dot(nf_r[...], w1n_r[...], precision=_PREC) + b1n_r[...], 0.0)
        nfo = jnp.dot(h, w2n_r[...], precision=_PREC) + b2n_r[...]
        nfo_o[...] = nfo
        p0_o[...] = jnp.dot(nfo, wb_r[...], precision=_PREC)
        p1_o[...] = jnp.dot(nfo, wc_r[...], precision=_PREC)
        wfe_o[...] = jnp.dot(w2e_r[...], wa_r[...], precision=_PREC)
        bfe_o[...] = jnp.dot(b2e_r[...], wa_r[...], precision=_PREC) + b1p_r[...]

    return pl.pallas_call(
        body,
        out_shape=[
            jax.ShapeDtypeStruct((_N, _H), _f32),
            jax.ShapeDtypeStruct((_N, _H), _f32),
            jax.ShapeDtypeStruct((_N, _H), _f32),
            jax.ShapeDtypeStruct((_H, _H), _f32),
            jax.ShapeDtypeStruct((1, _H), _f32),
        ],
    )(nf, w1n, b1n, w2n, b2n, wb, wc, w2e, wa, b2e, b1p)


def _sc_gather(p0, p1, i0, i1):
    mesh = plsc.VectorSubcoreMesh(core_axis_name="c", subcore_axis_name="s")

    def gather_k(p0_hbm, p1_hbm, i0_hbm, i1_hbm, g_hbm,
                 i0_v, i1_v, r0_v, r1_v, sg0, sg1, sw0, sw1):
        wid = lax.axis_index("s") * _NC + lax.axis_index("c")
        base = wid * _EPW
        sg = (sg0, sg1)
        sw = (sw0, sw1)

        def load_idx(q, b):
            off = base + q * _CH
            pltpu.sync_copy(i0_hbm.at[pl.ds(off, _CH)], i0_v.at[b])
            pltpu.sync_copy(i1_hbm.at[pl.ds(off, _CH)], i1_v.at[b])

        def start_gather(b):
            pltpu.async_copy(p0_hbm.at[i0_v.at[b]], r0_v.at[b], sg[b])
            pltpu.async_copy(p1_hbm.at[i1_v.at[b]], r1_v.at[b], sg[b])

        def wait_gather(b):
            pltpu.make_async_copy(p0_hbm.at[i0_v.at[b]], r0_v.at[b], sg[b]).wait()
            pltpu.make_async_copy(p1_hbm.at[i1_v.at[b]], r1_v.at[b], sg[b]).wait()

        def add_rows(b):
            def row(rr, carry):
                for cc in range(_H // 16):
                    sl = pl.ds(cc * 16, 16)
                    r0_v[b, rr, sl] = r0_v[b, rr, sl] + r1_v[b, rr, sl]
                return carry
            lax.fori_loop(0, _CH, row, 0)

        def start_wb(q, b):
            off = base + q * _CH
            pltpu.async_copy(r0_v.at[b], g_hbm.at[pl.ds(off, _CH)], sw[b])

        def wait_wb(q, b):
            off = base + q * _CH
            pltpu.make_async_copy(r0_v.at[b], g_hbm.at[pl.ds(off, _CH)], sw[b]).wait()

        load_idx(0, 0)
        start_gather(0)

        def pair(k, carry):
            for b in (0, 1):
                q = 2 * k + b
                nb = 1 - b
                load_idx(q + 1, nb)
                if b == 0:
                    @pl.when(k >= 1)
                    def _():
                        wait_wb(q - 1, nb)
                else:
                    wait_wb(q - 1, nb)
                start_gather(nb)
                wait_gather(b)
                add_rows(b)
                start_wb(q, b)
            return carry

        lax.fori_loop(0, (_NCHUNK - 1) // 2, pair, 0)
        qlast = _NCHUNK - 1
        wait_gather(0)
        add_rows(0)
        start_wb(qlast, 0)
        wait_wb(qlast - 1, 1)
        wait_wb(qlast, 0)

    call = pl.kernel(
        gather_k,
        mesh=mesh,
        out_type=jax.ShapeDtypeStruct((_E, _H), _f32),
        scratch_types=[
            pltpu.VMEM((2, _CH), jnp.int32),
            pltpu.VMEM((2, _CH), jnp.int32),
            pltpu.VMEM((2, _CH, _H), _f32),
            pltpu.VMEM((2, _CH, _H), _f32),
            pltpu.SemaphoreType.DMA,
            pltpu.SemaphoreType.DMA,
            pltpu.SemaphoreType.DMA,
            pltpu.SemaphoreType.DMA,
        ],
    )
    return call(p0, p1, i0, i1)


def _tc_edge(ef, g, w1e, b1e, wfe, bfe, w2p, b2p):
    nblk = _E // _BE

    def body(ef_r, g_r, w1e_r, b1e_r, wfe_r, bfe_r, w2p_r, b2p_r, m_o):
        h = jnp.maximum(jnp.dot(ef_r[...], w1e_r[...], precision=_EPREC) + b1e_r[...], 0.0)
        z = jnp.maximum(
            jnp.dot(h, wfe_r[...], precision=_EPREC) + bfe_r[...] + g_r[...], 0.0)
        m_o[...] = jnp.dot(z, w2p_r[...], precision=_EPREC) + b2p_r[...]

    def full(shape):
        return pl.BlockSpec(shape, lambda i: (0, 0))

    return pl.pallas_call(
        body,
        grid=(nblk,),
        in_specs=[
            pl.BlockSpec((_BE, _DE), lambda i: (i, 0)),
            pl.BlockSpec((_BE, _H), lambda i: (i, 0)),
            full((_DE, _H)),
            full((1, _H)),
            full((_H, _H)),
            full((1, _H)),
            full((_H, _H)),
            full((1, _H)),
        ],
        out_specs=pl.BlockSpec((_BE, _H), lambda i: (i, 0)),
        out_shape=jax.ShapeDtypeStruct((_E, _H), _f32),
    )(ef, g, w1e, b1e, wfe, bfe, w2p, b2p)


def _sc_scatter(m, i0, zeros):
    mesh = plsc.VectorSubcoreMesh(core_axis_name="c", subcore_axis_name="s")

    def scatter_k(m_hbm, i0_hbm, z_hbm, out_hbm, i_v, r_v, acc_sh, sl0, sl1):
        c = lax.axis_index("c")
        s = lax.axis_index("s")
        sl = (sl0, sl1)
        pltpu.sync_copy(z_hbm, acc_sh.at[pl.ds(s * _RPN, _RPN)])
        plsc.subcore_barrier()

        base = (c * _NS + s) * _EPW

        def start_load(q, b):
            off = base + q * _CH
            pltpu.async_copy(m_hbm.at[pl.ds(off, _CH)], r_v.at[b], sl[b])
            pltpu.async_copy(i0_hbm.at[pl.ds(off, _CH)], i_v.at[b], sl[b])

        def wait_load(q, b):
            off = base + q * _CH
            pltpu.make_async_copy(m_hbm.at[pl.ds(off, _CH)], r_v.at[b], sl[b]).wait()
            pltpu.make_async_copy(i0_hbm.at[pl.ds(off, _CH)], i_v.at[b], sl[b]).wait()

        def scat(b):
            pltpu.sync_copy(r_v.at[b], acc_sh.at[i_v.at[b]], add=True)

        start_load(0, 0)

        def pair(k, carry):
            for b in (0, 1):
                q = 2 * k + b
                start_load(q + 1, 1 - b)
                wait_load(q, b)
                scat(b)
            return carry

        lax.fori_loop(0, (_NCHUNK - 1) // 2, pair, 0)
        wait_load(_NCHUNK - 1, 0)
        scat(0)
        plsc.subcore_barrier()
        pltpu.sync_copy(acc_sh.at[pl.ds(s * _RPN, _RPN)],
                        out_hbm.at[pl.ds(c * _NPAD + s * _RPN, _RPN)])

    call = pl.kernel(
        scatter_k,
        mesh=mesh,
        out_type=jax.ShapeDtypeStruct((_NC * _NPAD, _H), _f32),
        scratch_types=[
            pltpu.VMEM((2, _CH), jnp.int32),
            pltpu.VMEM((2, _CH, _H), _f32),
            pltpu.VMEM_SHARED((_NPAD, _H), _f32),
            pltpu.SemaphoreType.DMA,
            pltpu.SemaphoreType.DMA,
        ],
    )
    return call(m, i0, zeros)


def _tc_node_post(nf, nfo, a0, a1, wqa, wqb, b1q, w2q, b2q, w1d, b1d, w2d, b2d):
    def body(nf_r, nfo_r, a0_r, a1_r, wqa_r, wqb_r, b1q_r, w2q_r, b2q_r,
             w1d_r, b1d_r, w2d_r, b2d_r, out_o):
        agg = a0_r[...] + a1_r[...]
        u = jnp.maximum(
            jnp.dot(nfo_r[...], wqa_r[...], precision=_PREC)
            + jnp.dot(agg, wqb_r[...], precision=_PREC) + b1q_r[...], 0.0)
        u = jnp.dot(u, w2q_r[...], precision=_PREC) + b2q_r[...]
        v = jnp.maximum(jnp.dot(u, w1d_r[...], precision=_PREC) + b1d_r[...], 0.0)
        v = jnp.dot(v, w2d_r[...], precision=_PREC) + b2d_r[...]
        out_o[...] = nf_r[...] + v

    return pl.pallas_call(
        body,
        out_shape=jax.ShapeDtypeStruct((_N, _DF), _f32),
    )(nf, nfo, a0, a1, wqa, wqb, b1q, w2q, b2q, w1d, b1d, w2d, b2d)


def kernel(edges, node_features, edge_features, node_enc, edge_enc, edge_proc,
           node_proc, dec):
    e0 = edges[0, 0].astype(jnp.int32)
    e1 = edges[0, 1].astype(jnp.int32)
    nf = node_features[0]
    ef = edge_features[0]
    w1n, b1n, w2n, b2n = node_enc
    w1e, b1e, w2e, b2e = edge_enc
    w1p, b1p, w2p, b2p = edge_proc
    w1q, b1q, w2q, b2q = node_proc
    w1d, b1d, w2d, b2d = dec

    wa, wb, wc = w1p[:_H], w1p[_H:2 * _H], w1p[2 * _H:]
    wqa, wqb = w1q[:_H], w1q[_H:]

    def r(b):
        return b.reshape(1, -1)

    nfo, p0, p1, wfe, bfe = _tc_node_pre(
        nf, w1n, r(b1n), w2n, r(b2n), wb, wc, w2e, wa, r(b2e), r(b1p))
    g = _sc_gather(p0, p1, e0, e1)
    m = _tc_edge(ef, g, w1e, r(b1e), wfe, bfe, w2p, r(b2p))
    zeros = jnp.zeros((_RPN, _H), _f32)
    aggp = _sc_scatter(m, e0, zeros)
    out = _tc_node_post(
        nf, nfo, aggp[:_N], aggp[_NPAD:_NPAD + _N], wqa, wqb, r(b1q), w2q, r(b2q),
        w1d, r(b1d), w2d, r(b2d))
    return out[None]
